# Initial kernel scaffold; baseline (speedup 1.0000x reference)
import jax
import jax.numpy as jnp
from jax.experimental import pallas as pl


def kernel(x, edge_index, edge_attr, adj, W_l1, b_l1, W_r1, b_r1, W_l2, b_l2, W_r2, b_r2, W_g1, b_g1, W_rel, b_rel, W_root, W_c0, W_c1, W_c2, b_c):
    raise NotImplementedError("write your pallas kernel here")


# speedup vs baseline: 25.2730x; 25.2730x over previous
import functools

import jax
import jax.numpy as jnp
from jax import lax
from jax.experimental import pallas as pl

N = 100
E = 4000
KPOOL = 50
NCLUST = 50
NEG_SLOPE = 0.01

_HI = lax.Precision.HIGHEST
_DF = lax.Precision.DEFAULT


def _dot(a, b, dims, prec):
    return lax.dot_general(a, b, (dims, ((), ())),
                           precision=prec, preferred_element_type=jnp.float32)


def _mm(a, b, prec):
    return _dot(a, b, ((1,), (0,)), prec)


def _leaky(v):
    return jnp.where(v >= 0, v, NEG_SLOPE * v)


def _softmax(v):
    m = jnp.max(v, axis=-1, keepdims=True)
    e = jnp.exp(v - m)
    return e / jnp.sum(e, axis=-1, keepdims=True)


def _fwd_kernel(row_ref, col_ref, ea_ref, x_ref,
                wl1_ref, bl1_ref, wr1_ref, br1_ref,
                wl2_ref, bl2_ref, wr2_ref, br2_ref,
                wg1_ref, bg1_ref, wrel_ref, brel_ref, wroot_ref,
                wc0_ref, wc1_ref, wc2_ref, bc_ref,
                out_ref):
    f32 = jnp.float32
    rowv = row_ref[...]
    colv = col_ref[...]
    ea = ea_ref[...]

    n_iota = lax.broadcasted_iota(jnp.int32, (E, N), 1)
    ohr = (rowv == n_iota).astype(f32)
    ohc = (colv == n_iota).astype(f32)
    lm = ((rowv < KPOOL) & (colv < KPOOL)).astype(f32)
    rm = ((rowv >= KPOOL) & (colv >= KPOOL)).astype(f32)
    stacked = jnp.concatenate(
        [ohr * ea, ohr * (ea * lm), ohr * (ea * rm), ohr], axis=1)
    a4 = _dot(ohc, stacked, ((0,), (0,)), _HI)

    eye = (lax.broadcasted_iota(jnp.int32, (N, N), 0)
           == lax.broadcasted_iota(jnp.int32, (N, N), 1)).astype(f32)
    a_f = a4[:, 0:N] + eye
    a_l = a4[:, N:2 * N] + eye
    a_r = a4[:, 2 * N:3 * N] + eye
    bc_mat = a4[:, 3 * N:4 * N]

    def inv_sqrt_deg(a):
        deg = jnp.sum(a, axis=1, keepdims=True)
        return lax.rsqrt(deg)

    dis_f = inv_sqrt_deg(a_f)
    dis_l = inv_sqrt_deg(a_l)
    dis_r = inv_sqrt_deg(a_r)

    def gcn(xw, a, dis, b):
        return dis * _mm(a, dis * xw, _HI) + b

    x = x_ref[...]
    riota64 = lax.broadcasted_iota(jnp.int32, (N, 64), 0)
    riota20 = lax.broadcasted_iota(jnp.int32, (N, 20), 0)

    hl = _leaky(gcn(_mm(x, wl1_ref[...], _DF), a_l, dis_l, bl1_ref[...]))
    hr = _leaky(gcn(_mm(x, wr1_ref[...], _DF), a_r, dis_r, br1_ref[...]))
    h1 = jnp.where(riota64 < KPOOL, hl, hr)

    hl2 = _leaky(gcn(_mm(h1, wl2_ref[...], _DF), a_l, dis_l, bl2_ref[...]))
    hr2 = _leaky(gcn(_mm(h1, wr2_ref[...], _DF), a_r, dis_r, br2_ref[...]))
    h2a = jnp.where(riota20 < KPOOL, hl2, hr2)

    h2 = _leaky(gcn(_mm(h2a, wg1_ref[...], _DF), a_f, dis_f, bg1_ref[...]))

    agg = _mm(bc_mat, h2, _HI)
    score = jnp.tanh(_mm(agg, wrel_ref[...], _DF) + brel_ref[...]
                     + _mm(h2, wroot_ref[...], _DF))

    score_row = _dot(score, eye, ((0,), (0,)), _HI)
    ri = lax.broadcasted_iota(jnp.int32, (N, N), 0)
    rj = lax.broadcasted_iota(jnp.int32, (N, N), 1)
    beats = ((score_row > score) |
             ((score_row == score) & (rj < ri))).astype(f32)
    rank = jnp.sum(beats, axis=1, keepdims=True)
    rank_row = _dot(rank, eye, ((0,), (0,)), _HI)

    piota = lax.broadcasted_iota(jnp.int32, (KPOOL, N), 0).astype(f32)
    perm_mat = (rank_row == piota).astype(f32)

    vals = _mm(perm_mat, score, _HI)
    x_pool = _mm(perm_mat, h2, _HI) * vals

    craw = _dot(_mm(perm_mat, bc_mat, _HI), perm_mat, ((1,), (1,)), _HI)
    deg_c = jnp.sum(craw, axis=1, keepdims=True)
    dis_c = jnp.where(deg_c > 0, lax.rsqrt(jnp.where(deg_c > 0, deg_c, 1.0)), 0.0)

    def prop_top(z):
        return -(dis_c * _mm(craw, dis_c * z, _HI))

    h2_top = h2[0:KPOOL, :]
    h2_bot = h2[KPOOL:N, :]
    t1_top = prop_top(h2_top)
    t2_top = 2.0 * prop_top(t1_top) - h2_top
    zeros_bot = jnp.zeros((N - KPOOL, 20), f32)
    tx1 = jnp.concatenate([t1_top, zeros_bot], axis=0)
    tx2 = jnp.concatenate([t2_top, -h2_bot], axis=0)

    cheb = (_mm(h2, wc0_ref[...], _DF) + _mm(tx1, wc1_ref[...], _DF)
            + _mm(tx2, wc2_ref[...], _DF) + bc_ref[...])
    ass = _softmax(cheb)
    s = _softmax(ass)

    h_coarse = _dot(s, h2, ((0,), (0,)), _DF)

    kept = (rank < KPOOL).astype(f32)
    tri = (rj < ri).astype(f32)
    cum_excl = _mm(tri, kept, _HI)
    cum_row = _dot(cum_excl, eye, ((0,), (0,)), _HI)
    kept_row = rank_row < KPOOL
    qiota = lax.broadcasted_iota(jnp.int32, (KPOOL, N), 0).astype(f32)
    q_mat = ((cum_row == qiota) & kept_row).astype(f32)

    inter = _mm(q_mat, ass, _HI)
    h1_out = _mm(inter, h_coarse, _DF)
    out_ref[...] = x_pool + h1_out


def kernel(x, edge_index, edge_attr, adj, W_l1, b_l1, W_r1, b_r1, W_l2, b_l2,
           W_r2, b_r2, W_g1, b_g1, W_rel, b_rel, W_root, W_c0, W_c1, W_c2, b_c,
           interpret=False):
    del adj
    row = edge_index[0].reshape(E, 1)
    col = edge_index[1].reshape(E, 1)
    ea = edge_attr.reshape(E, 1).astype(jnp.float32)
    operands = (
        row, col, ea, x,
        W_l1, b_l1.reshape(1, -1), W_r1, b_r1.reshape(1, -1),
        W_l2, b_l2.reshape(1, -1), W_r2, b_r2.reshape(1, -1),
        W_g1, b_g1.reshape(1, -1), W_rel, b_rel.reshape(1, 1), W_root,
        W_c0, W_c1, W_c2, b_c.reshape(1, -1),
    )
    h2_out = pl.pallas_call(
        _fwd_kernel,
        out_shape=jax.ShapeDtypeStruct((KPOOL, 20), jnp.float32),
        interpret=interpret,
    )(*operands)
    return h2_out.reshape(1, -1)


# speedup vs baseline: 27.7151x; 1.0966x over previous
import functools

import jax
import jax.numpy as jnp
from jax import lax
from jax.experimental import pallas as pl

N = 100
E = 4000
KPOOL = 50
NCLUST = 50
NEG_SLOPE = 0.01

_HI = lax.Precision.HIGHEST
_DF = lax.Precision.DEFAULT


def _dot(a, b, dims, prec):
    return lax.dot_general(a, b, (dims, ((), ())),
                           precision=prec, preferred_element_type=jnp.float32)


def _mm(a, b, prec):
    return _dot(a, b, ((1,), (0,)), prec)


def _leaky(v):
    return jnp.where(v >= 0, v, NEG_SLOPE * v)


def _softmax(v):
    m = jnp.max(v, axis=-1, keepdims=True)
    e = jnp.exp(v - m)
    return e / jnp.sum(e, axis=-1, keepdims=True)


def _fwd_kernel(row_ref, col_ref, ea_ref, x_ref,
                wl1_ref, bl1_ref, wr1_ref, br1_ref,
                wl2_ref, bl2_ref, wr2_ref, br2_ref,
                wg1_ref, bg1_ref, wrel_ref, brel_ref, wroot_ref,
                wc0_ref, wc1_ref, wc2_ref, bc_ref,
                out_ref):
    f32 = jnp.float32
    rowv = row_ref[...]
    colv = col_ref[...]
    ea = ea_ref[...]

    bf16 = jnp.bfloat16
    n_iota = lax.broadcasted_iota(jnp.int32, (E, N), 1)
    ohr = (rowv == n_iota).astype(bf16)
    ohc = (colv == n_iota).astype(bf16)
    ea_hi = ea.astype(bf16)
    ea_mid = (ea - ea_hi.astype(f32)).astype(bf16)
    ea_lo = (ea - ea_hi.astype(f32) - ea_mid.astype(f32)).astype(bf16)
    stacked = jnp.concatenate(
        [ohr * ea_hi, ohr * ea_mid, ohr * ea_lo, ohr], axis=1)
    a4 = _dot(ohc, stacked, ((0,), (0,)), _DF)

    eye = (lax.broadcasted_iota(jnp.int32, (N, N), 0)
           == lax.broadcasted_iota(jnp.int32, (N, N), 1)).astype(f32)
    a_w = a4[:, 0:N] + a4[:, N:2 * N] + a4[:, 2 * N:3 * N]
    bc_mat = a4[:, 3 * N:4 * N]
    a_f = a_w + eye
    ci = lax.broadcasted_iota(jnp.int32, (N, N), 0)
    ri = lax.broadcasted_iota(jnp.int32, (N, N), 1)
    in_l = (ci < KPOOL) & (ri < KPOOL)
    in_r = (ci >= KPOOL) & (ri >= KPOOL)
    a_l = jnp.where(in_l, a_w, 0.0) + eye
    a_r = jnp.where(in_r, a_w, 0.0) + eye

    def inv_sqrt_deg(a):
        deg = jnp.sum(a, axis=1, keepdims=True)
        return lax.rsqrt(deg)

    dis_f = inv_sqrt_deg(a_f)
    dis_l = inv_sqrt_deg(a_l)
    dis_r = inv_sqrt_deg(a_r)

    def gcn(xw, a, dis, b):
        return dis * _mm(a, dis * xw, _HI) + b

    x = x_ref[...]
    riota64 = lax.broadcasted_iota(jnp.int32, (N, 64), 0)
    riota20 = lax.broadcasted_iota(jnp.int32, (N, 20), 0)

    hl = _leaky(gcn(_mm(x, wl1_ref[...], _DF), a_l, dis_l, bl1_ref[...]))
    hr = _leaky(gcn(_mm(x, wr1_ref[...], _DF), a_r, dis_r, br1_ref[...]))
    h1 = jnp.where(riota64 < KPOOL, hl, hr)

    hl2 = _leaky(gcn(_mm(h1, wl2_ref[...], _DF), a_l, dis_l, bl2_ref[...]))
    hr2 = _leaky(gcn(_mm(h1, wr2_ref[...], _DF), a_r, dis_r, br2_ref[...]))
    h2a = jnp.where(riota20 < KPOOL, hl2, hr2)

    h2 = _leaky(gcn(_mm(h2a, wg1_ref[...], _DF), a_f, dis_f, bg1_ref[...]))

    agg = _mm(bc_mat, h2, _HI)
    score = jnp.tanh(_mm(agg, wrel_ref[...], _DF) + brel_ref[...]
                     + _mm(h2, wroot_ref[...], _DF))

    score_row = _dot(score, eye, ((0,), (0,)), _HI)
    ri = lax.broadcasted_iota(jnp.int32, (N, N), 0)
    rj = lax.broadcasted_iota(jnp.int32, (N, N), 1)
    beats = ((score_row > score) |
             ((score_row == score) & (rj < ri))).astype(f32)
    rank = jnp.sum(beats, axis=1, keepdims=True)
    rank_row = _dot(rank, eye, ((0,), (0,)), _HI)

    piota = lax.broadcasted_iota(jnp.int32, (KPOOL, N), 0).astype(f32)
    perm_mat = (rank_row == piota).astype(f32)

    vals = _mm(perm_mat, score, _HI)
    x_pool = _mm(perm_mat, h2, _HI) * vals

    craw = _dot(_mm(perm_mat, bc_mat, _HI), perm_mat, ((1,), (1,)), _HI)
    deg_c = jnp.sum(craw, axis=1, keepdims=True)
    dis_c = jnp.where(deg_c > 0, lax.rsqrt(jnp.where(deg_c > 0, deg_c, 1.0)), 0.0)

    def prop_top(z):
        return -(dis_c * _mm(craw, dis_c * z, _HI))

    h2_top = h2[0:KPOOL, :]
    h2_bot = h2[KPOOL:N, :]
    t1_top = prop_top(h2_top)
    t2_top = 2.0 * prop_top(t1_top) - h2_top
    zeros_bot = jnp.zeros((N - KPOOL, 20), f32)
    tx1 = jnp.concatenate([t1_top, zeros_bot], axis=0)
    tx2 = jnp.concatenate([t2_top, -h2_bot], axis=0)

    cheb = (_mm(h2, wc0_ref[...], _DF) + _mm(tx1, wc1_ref[...], _DF)
            + _mm(tx2, wc2_ref[...], _DF) + bc_ref[...])
    ass = _softmax(cheb)
    s = _softmax(ass)

    h_coarse = _dot(s, h2, ((0,), (0,)), _DF)

    kept = (rank < KPOOL).astype(f32)
    tri = (rj < ri).astype(f32)
    cum_excl = _mm(tri, kept, _HI)
    cum_row = _dot(cum_excl, eye, ((0,), (0,)), _HI)
    kept_row = rank_row < KPOOL
    qiota = lax.broadcasted_iota(jnp.int32, (KPOOL, N), 0).astype(f32)
    q_mat = ((cum_row == qiota) & kept_row).astype(f32)

    inter = _mm(q_mat, ass, _HI)
    h1_out = _mm(inter, h_coarse, _DF)
    out_ref[...] = x_pool + h1_out


def kernel(x, edge_index, edge_attr, adj, W_l1, b_l1, W_r1, b_r1, W_l2, b_l2,
           W_r2, b_r2, W_g1, b_g1, W_rel, b_rel, W_root, W_c0, W_c1, W_c2, b_c,
           interpret=False):
    del adj
    row = edge_index[0].reshape(E, 1)
    col = edge_index[1].reshape(E, 1)
    ea = edge_attr.reshape(E, 1).astype(jnp.float32)
    operands = (
        row, col, ea, x,
        W_l1, b_l1.reshape(1, -1), W_r1, b_r1.reshape(1, -1),
        W_l2, b_l2.reshape(1, -1), W_r2, b_r2.reshape(1, -1),
        W_g1, b_g1.reshape(1, -1), W_rel, b_rel.reshape(1, 1), W_root,
        W_c0, W_c1, W_c2, b_c.reshape(1, -1),
    )
    h2_out = pl.pallas_call(
        _fwd_kernel,
        out_shape=jax.ShapeDtypeStruct((KPOOL, 20), jnp.float32),
        interpret=interpret,
    )(*operands)
    return h2_out.reshape(1, -1)


# speedup vs baseline: 52.6075x; 1.8982x over previous
import jax
import jax.numpy as jnp
from jax import lax
from jax.experimental import pallas as pl

N = 100
E = 4000
KPOOL = 50
NCLUST = 50
NEG_SLOPE = 0.01

_HI = lax.Precision.HIGHEST
_DF = lax.Precision.DEFAULT


def _dot(a, b, dims, prec):
    return lax.dot_general(a, b, (dims, ((), ())),
                           precision=prec, preferred_element_type=jnp.float32)


def _mm(a, b, prec):
    return _dot(a, b, ((1,), (0,)), prec)


def _leaky(v):
    return jnp.where(v >= 0, v, NEG_SLOPE * v)


def _softmax(v):
    m = jnp.max(v, axis=-1, keepdims=True)
    e = jnp.exp(v - m)
    return e / jnp.sum(e, axis=-1, keepdims=True)


def _fwd_kernel(ei_ref, ea_ref, x_ref,
                wl1_ref, bl1_ref, wr1_ref, br1_ref,
                wl2_ref, bl2_ref, wr2_ref, br2_ref,
                wg1_ref, bg1_ref, wrel_ref, brel_ref, wroot_ref,
                wc0_ref, wc1_ref, wc2_ref, bc_ref,
                out_ref):
    f32 = jnp.float32
    rowv = ei_ref[0:1, :]
    colv = ei_ref[1:2, :]
    ea = ea_ref[...]

    bf16 = jnp.bfloat16
    n_iota = lax.broadcasted_iota(jnp.int32, (N, E), 0)
    ohr_t = (rowv == n_iota).astype(bf16)
    ohc_t = (colv == n_iota).astype(bf16)
    ea_hi = ea.astype(bf16)
    ea_mid = (ea - ea_hi.astype(f32)).astype(bf16)
    ea_lo = (ea - ea_hi.astype(f32) - ea_mid.astype(f32)).astype(bf16)
    stacked_t = jnp.concatenate(
        [ohr_t * ea_hi, ohr_t * ea_mid, ohr_t * ea_lo, ohr_t], axis=0)
    a4 = _dot(ohc_t, stacked_t, ((1,), (1,)), _DF)

    eye = (lax.broadcasted_iota(jnp.int32, (N, N), 0)
           == lax.broadcasted_iota(jnp.int32, (N, N), 1)).astype(f32)
    a_w = a4[:, 0:N] + a4[:, N:2 * N] + a4[:, 2 * N:3 * N]
    bc_mat = a4[:, 3 * N:4 * N]
    a_f = a_w + eye
    ci = lax.broadcasted_iota(jnp.int32, (N, N), 0)
    ri = lax.broadcasted_iota(jnp.int32, (N, N), 1)
    in_l = (ci < KPOOL) & (ri < KPOOL)
    in_r = (ci >= KPOOL) & (ri >= KPOOL)
    a_l = jnp.where(in_l, a_w, 0.0) + eye
    a_r = jnp.where(in_r, a_w, 0.0) + eye

    def inv_sqrt_deg(a):
        deg = jnp.sum(a, axis=1, keepdims=True)
        return lax.rsqrt(deg)

    dis_f = inv_sqrt_deg(a_f)
    dis_l = inv_sqrt_deg(a_l)
    dis_r = inv_sqrt_deg(a_r)

    def gcn(xw, a, dis, b):
        return dis * _mm(a, dis * xw, _HI) + b

    x = x_ref[...]
    riota64 = lax.broadcasted_iota(jnp.int32, (N, 64), 0)
    riota20 = lax.broadcasted_iota(jnp.int32, (N, 20), 0)

    hl = _leaky(gcn(_mm(x, wl1_ref[...], _DF), a_l, dis_l, bl1_ref[...]))
    hr = _leaky(gcn(_mm(x, wr1_ref[...], _DF), a_r, dis_r, br1_ref[...]))
    h1 = jnp.where(riota64 < KPOOL, hl, hr)

    hl2 = _leaky(gcn(_mm(h1, wl2_ref[...], _DF), a_l, dis_l, bl2_ref[...]))
    hr2 = _leaky(gcn(_mm(h1, wr2_ref[...], _DF), a_r, dis_r, br2_ref[...]))
    h2a = jnp.where(riota20 < KPOOL, hl2, hr2)

    h2 = _leaky(gcn(_mm(h2a, wg1_ref[...], _DF), a_f, dis_f, bg1_ref[...]))

    agg = _mm(bc_mat, h2, _HI)
    score = jnp.tanh(_mm(agg, wrel_ref[...], _DF) + brel_ref[...]
                     + _mm(h2, wroot_ref[...], _DF))

    score_row = _dot(score, eye, ((0,), (0,)), _HI)
    ri = lax.broadcasted_iota(jnp.int32, (N, N), 0)
    rj = lax.broadcasted_iota(jnp.int32, (N, N), 1)
    beats = ((score_row > score) |
             ((score_row == score) & (rj < ri))).astype(f32)
    rank = jnp.sum(beats, axis=1, keepdims=True)
    rank_row = _dot(rank, eye, ((0,), (0,)), _HI)

    piota = lax.broadcasted_iota(jnp.int32, (KPOOL, N), 0).astype(f32)
    perm_mat = (rank_row == piota).astype(f32)

    vals = _mm(perm_mat, score, _HI)
    x_pool = _mm(perm_mat, h2, _HI) * vals

    craw = _dot(_mm(perm_mat, bc_mat, _HI), perm_mat, ((1,), (1,)), _HI)
    deg_c = jnp.sum(craw, axis=1, keepdims=True)
    dis_c = jnp.where(deg_c > 0, lax.rsqrt(jnp.where(deg_c > 0, deg_c, 1.0)), 0.0)

    def prop_top(z):
        return -(dis_c * _mm(craw, dis_c * z, _HI))

    h2_top = h2[0:KPOOL, :]
    h2_bot = h2[KPOOL:N, :]
    t1_top = prop_top(h2_top)
    t2_top = 2.0 * prop_top(t1_top) - h2_top
    zeros_bot = jnp.zeros((N - KPOOL, 20), f32)
    tx1 = jnp.concatenate([t1_top, zeros_bot], axis=0)
    tx2 = jnp.concatenate([t2_top, -h2_bot], axis=0)

    cheb = (_mm(h2, wc0_ref[...], _DF) + _mm(tx1, wc1_ref[...], _DF)
            + _mm(tx2, wc2_ref[...], _DF) + bc_ref[...])
    ass = _softmax(cheb)
    s = _softmax(ass)

    h_coarse = _dot(s, h2, ((0,), (0,)), _DF)

    kept = (rank < KPOOL).astype(f32)
    tri = (rj < ri).astype(f32)
    cum_excl = _mm(tri, kept, _HI)
    cum_row = _dot(cum_excl, eye, ((0,), (0,)), _HI)
    kept_row = rank_row < KPOOL
    qiota = lax.broadcasted_iota(jnp.int32, (KPOOL, N), 0).astype(f32)
    q_mat = ((cum_row == qiota) & kept_row).astype(f32)

    inter = _mm(q_mat, ass, _HI)
    h1_out = _mm(inter, h_coarse, _DF)
    out_ref[...] = x_pool + h1_out


def kernel(x, edge_index, edge_attr, adj, W_l1, b_l1, W_r1, b_r1, W_l2, b_l2,
           W_r2, b_r2, W_g1, b_g1, W_rel, b_rel, W_root, W_c0, W_c1, W_c2, b_c,
           interpret=False):
    del adj
    ea = edge_attr.reshape(1, E).astype(jnp.float32)
    operands = (
        edge_index, ea, x,
        W_l1, b_l1.reshape(1, -1), W_r1, b_r1.reshape(1, -1),
        W_l2, b_l2.reshape(1, -1), W_r2, b_r2.reshape(1, -1),
        W_g1, b_g1.reshape(1, -1), W_rel, b_rel.reshape(1, 1), W_root,
        W_c0, W_c1, W_c2, b_c.reshape(1, -1),
    )
    h2_out = pl.pallas_call(
        _fwd_kernel,
        out_shape=jax.ShapeDtypeStruct((KPOOL, 20), jnp.float32),
        interpret=interpret,
    )(*operands)
    return h2_out.reshape(1, -1)
